# baseline (speedup 1.0000x reference)
import functools

import jax
import jax.numpy as jnp
from jax import lax
from jax.experimental import pallas as pl
from jax.experimental.pallas import tpu as pltpu
from jax.experimental.pallas import tpu_sc as plsc

NC = 2
NS = 16
NW = NC * NS
CH = 128


@jax.jit
def _sc_gather(table, idx_grid):
  NWg, n_ch, _ = idx_grid.shape
  _, D = table.shape
  mesh = plsc.VectorSubcoreMesh(
      core_axis_name="c", subcore_axis_name="s", num_cores=NC,
      num_subcores=NS)

  @functools.partial(
      pl.kernel,
      out_type=jax.ShapeDtypeStruct((NWg, n_ch, CH, D), jnp.float32),
      mesh=mesh,
      scratch_types=[
          pltpu.VMEM((n_ch, CH), jnp.int32),
          pltpu.VMEM((n_ch, CH, D), jnp.float32),
          pltpu.SemaphoreType.DMA,
      ],
      compiler_params=pltpu.CompilerParams(use_tc_tiling_on_sc=False),
  )
  def body(table_hbm, idx_hbm, out_hbm, idx_v, rows_v, sem):
    wid = lax.axis_index("s") * NC + lax.axis_index("c")
    pltpu.sync_copy(idx_hbm.at[wid], idx_v)
    copies = [
        pltpu.async_copy(table_hbm.at[idx_v.at[j]], rows_v.at[j], sem)
        for j in range(n_ch)
    ]
    for c in copies:
      c.wait()
    pltpu.sync_copy(rows_v, out_hbm.at[wid])

  return body(table, idx_grid)


def kernel(ordered_embs, nodes):
  V, D = ordered_embs.shape
  B = nodes.shape[0]
  b_per_w = B // NW
  n_ch = b_per_w // CH
  idx_grid = nodes.reshape(NW, n_ch, CH)
  out = _sc_gather(ordered_embs, idx_grid)
  return out.reshape(B, D)
